# baseline (speedup 1.0000x reference)
import functools

import jax
import jax.numpy as jnp
from jax import lax
from jax.experimental import pallas as pl
from jax.experimental.pallas import tpu as pltpu
from jax.experimental.pallas import tpu_sc as plsc

N = 10000
E = 320000
F_IN = 128
H = 16
C = 64
NCLASS = 10
NGRAPH = 64

NSC = 2
NSUB = 16
ES = E // NSC
PAD = 1056
DPT = 624
DG = 8
EC = 16
RPTR_W = N + 64
SBLK = 1040
TW = 1152
CLAMP = 60.0


def _sc_layer_body(xl_hbm, xr_hbm, att_hbm, srcs_hbm, rptr_hbm,
                   num_hbm, den_hbm,
                   attv, rptrv, xrg, xjb0, xjb1, xjb2, xjb3, outg, deng, sblk,
                   gsem0, gsem1, gsem2, gsem3,
                   interp=False):
    if interp:
        s, w = pl.program_id(0), pl.program_id(1)
    else:
        s = lax.axis_index("c")
        w = lax.axis_index("s")

    pltpu.sync_copy(att_hbm, attv)

    tile_d0 = w * DPT
    ngrp = jnp.where(w == NSUB - 1, (N - DPT * (NSUB - 1)) // DG, DPT // DG)
    pltpu.sync_copy(rptr_hbm.at[pl.ds(s * RPTR_W + tile_d0, 656)], rptrv)
    src_base = s * (ES + PAD)

    def group_body(g, _):
        d0 = tile_d0 + g * DG
        pltpu.sync_copy(xr_hbm.at[pl.ds(d0, DG)], xrg)

        zero = jnp.zeros((16,), jnp.float32)

        def zero_dl(dl, _):
            deng[dl, :] = zero

            def zero_c(c, _):
                outg[dl, pl.ds(c * H, H)] = zero
                return 0
            lax.fori_loop(0, C, zero_c, 0)
            return 0
        lax.fori_loop(0, DG, zero_dl, 0)

        def seg_at(dl):
            dlc = jnp.minimum(dl, DG)
            rv = rptrv[pl.ds(g * DG + dlc, 16)]
            ss = rv[0]
            ln = jnp.where(dl < DG, rv[1] - ss, 0)
            return ss, ln

        def valid(st):
            dl, k, ss, ln = st
            return (dl < DG) & (k * EC < ln)

        def advance(st):
            dl, k, ss, ln = st
            more = (k + 1) * EC < ln
            dl2 = jnp.where(more, dl, jnp.minimum(dl + 1, DG))
            k2 = jnp.where(more, k + 1, 0)
            ss2, ln2 = seg_at(dl2)
            ss2 = jnp.where(more, ss, ss2)
            ln2 = jnp.where(more, ln, ln2)
            return (dl2, k2, ss2, ln2)

        def issue(st, blk_lo, buf, sem):
            dl, k, ss, ln = st
            v = valid(st)
            cs = ss + k * EC
            need = v & ((cs < blk_lo) | (cs + EC > blk_lo + SBLK))
            cs8 = (cs // 8) * 8
            blk_lo2 = jnp.where(need, cs8, blk_lo)

            @pl.when(need)
            def _():
                pltpu.sync_copy(srcs_hbm.at[pl.ds(src_base + cs8, SBLK)],
                                sblk)

            @pl.when(v)
            def _():
                srcv = sblk[pl.ds(cs - blk_lo2, EC)]
                pltpu.async_copy(xl_hbm.at[srcv], buf, sem)
            return blk_lo2

        def compute(st, buf, sem):
            v = valid(st)

            @pl.when(v)
            def _():
                pltpu.make_async_copy(xl_hbm.at[pl.ds(0, EC)], buf, sem).wait()

            if True:
                dl, k, ss, ln = st
                dl = jnp.minimum(dl, DG - 1)
                accs = [jnp.zeros((16,), jnp.float32) for _ in range(EC)]

                def chan_body(c, accs):
                    ch = pl.ds(c * H, H)
                    att_c = attv[ch]
                    xi_c = xrg[dl, ch]
                    out = []
                    for e in range(EC):
                        u = xi_c + buf[e, ch]
                        out.append(accs[e] + att_c * jnp.abs(u))
                    return out
                accs = lax.fori_loop(0, C, chan_body, accs)

                arv = xrg[dl, pl.ds(C * H, H)]
                rem = ln - k * EC
                ps = []
                for e in range(EC):
                    l_e = accs[e] + (buf[e, pl.ds(C * H, H)] + arv)
                    l_e = jnp.minimum(jnp.maximum(l_e, -CLAMP), CLAMP)
                    p_e = jnp.exp(l_e)
                    ps.append(jnp.where(rem > e, p_e, jnp.zeros_like(p_e)))
                dsum = ps[0]
                for e in range(1, EC):
                    dsum = dsum + ps[e]
                deng[dl, :] = deng[dl, :] + dsum

                def agg_body(c, _):
                    ch = pl.ds(c * H, H)
                    o = outg[dl, ch]
                    for e in range(EC):
                        o = o + ps[e] * buf[e, ch]
                    outg[dl, ch] = o
                    return 0
                lax.fori_loop(0, C, agg_body, 0)

        def trip_body(dl, t):
            _, ln = seg_at(dl)
            return t + jnp.maximum((ln + EC - 1) // EC, 1)

        trip = lax.fori_loop(0, DG, trip_body, jnp.int32(0))

        bufs = (xjb0, xjb1, xjb2, xjb3)
        sems = (gsem0, gsem1, gsem2, gsem3)
        ss0, ln0 = seg_at(jnp.int32(0))
        st = (jnp.int32(0), jnp.int32(0), ss0, ln0)
        blk_lo = jnp.int32(-2**30)
        sts = []
        for r in range(4):
            sts.append(st)
            blk_lo = issue(st, blk_lo, bufs[r], sems[r])
            st = advance(st)

        def body(q, carry):
            st0, st1, st2, st3, stn, blk_lo = carry
            sts = [st0, st1, st2, st3]
            for r in range(4):
                compute(sts[r], bufs[r], sems[r])
                blk_lo = issue(stn, blk_lo, bufs[r], sems[r])
                sts[r] = stn
                stn = advance(stn)
            return (sts[0], sts[1], sts[2], sts[3], stn, blk_lo)

        lax.fori_loop(0, (trip + 3) // 4, body,
                      (sts[0], sts[1], sts[2], sts[3], st, blk_lo))

        pltpu.sync_copy(outg, num_hbm.at[pl.ds(s * N + d0, DG)])
        pltpu.sync_copy(deng, den_hbm.at[pl.ds(s * N + d0, DG)])
        return 0

    lax.fori_loop(0, ngrp, group_body, 0)


@functools.partial(jax.jit, static_argnames=("interpret",))
def _sc_layer(xl, xr, attp, srcs, rptr, interpret=False):
    mesh = plsc.VectorSubcoreMesh(core_axis_name="c", subcore_axis_name="s",
                                  num_cores=NSC, num_subcores=NSUB)
    f = pl.kernel(
        functools.partial(_sc_layer_body, interp=interpret),
        out_type=[
            jax.ShapeDtypeStruct((NSC * N, C * H), jnp.float32),
            jax.ShapeDtypeStruct((NSC * N, H), jnp.float32),
        ],
        mesh=mesh,
        scratch_types=[
            pltpu.VMEM((C * H,), jnp.float32),
            pltpu.VMEM((656,), jnp.int32),
            pltpu.VMEM((DG, TW), jnp.float32),
            pltpu.VMEM((EC, TW), jnp.float32),
            pltpu.VMEM((EC, TW), jnp.float32),
            pltpu.VMEM((EC, TW), jnp.float32),
            pltpu.VMEM((EC, TW), jnp.float32),
            pltpu.VMEM((DG, C * H), jnp.float32),
            pltpu.VMEM((DG, H), jnp.float32),
            pltpu.VMEM((SBLK,), jnp.int32),
            pltpu.SemaphoreType.DMA,
            pltpu.SemaphoreType.DMA,
            pltpu.SemaphoreType.DMA,
            pltpu.SemaphoreType.DMA,
        ],
        interpret=interpret,
    )
    return f(xl, xr, attp, srcs, rptr)


BN = 400
TGRID = N // BN


def _adot_table(v, att_row):
    rows = jax.lax.broadcasted_iota(jnp.int32, (H * C, H), 0) % H
    colh = jax.lax.broadcasted_iota(jnp.int32, (H * C, H), 1)
    T2 = (rows == colh).astype(jnp.float32)
    a = jnp.dot(v * att_row, T2, preferred_element_type=jnp.float32) * 0.6
    pad = jnp.zeros((v.shape[0], TW - H * C - H), jnp.float32)
    return jnp.concatenate([v, a, pad], axis=1)


def _tc1_body(x_ref, wl_ref, bl_ref, wr_ref, br_ref, att_ref,
              xl_ref, xr_ref):
    x = x_ref[...]
    att_row = att_ref[...]
    xl = jnp.dot(x, wl_ref[...],
                 preferred_element_type=jnp.float32) + bl_ref[...]
    xr = jnp.dot(x, wr_ref[...],
                 preferred_element_type=jnp.float32) + br_ref[...]
    xl_ref[...] = _adot_table(xl, att_row)
    xr_ref[...] = _adot_table(xr, att_row)


def _tc1(x, Wlp, blp, Wrp, brp, att_row):
    f_in = x.shape[1]
    return pl.pallas_call(
        _tc1_body,
        grid=(TGRID,),
        in_specs=[
            pl.BlockSpec((BN, f_in), lambda i: (i, 0)),
            pl.BlockSpec((f_in, H * C), lambda i: (0, 0)),
            pl.BlockSpec((1, H * C), lambda i: (0, 0)),
            pl.BlockSpec((f_in, H * C), lambda i: (0, 0)),
            pl.BlockSpec((1, H * C), lambda i: (0, 0)),
            pl.BlockSpec((1, H * C), lambda i: (0, 0)),
        ],
        out_specs=[
            pl.BlockSpec((BN, TW), lambda i: (i, 0)),
            pl.BlockSpec((BN, TW), lambda i: (i, 0)),
        ],
        out_shape=[
            jax.ShapeDtypeStruct((N, TW), jnp.float32),
            jax.ShapeDtypeStruct((N, TW), jnp.float32),
        ],
    )(x, Wlp, blp[None, :], Wrp, brp[None, :], att_row[None, :])


def _combine_h(n0, n1, d0, d1, bias):
    dsum = d0 + d1 + 1e-16
    col = jax.lax.broadcasted_iota(jnp.int32, (H, H * C), 1) % H
    rowh = jax.lax.broadcasted_iota(jnp.int32, (H, H * C), 0)
    T = (col == rowh).astype(jnp.float32)
    den_b = jnp.dot(dsum, T, preferred_element_type=jnp.float32)
    hfull = (n0 + n1) / den_b
    rows = jax.lax.broadcasted_iota(jnp.int32, (H * C, C), 0) // H
    colc = jax.lax.broadcasted_iota(jnp.int32, (H * C, C), 1)
    S = (rows == colc).astype(jnp.float32)
    hm = jnp.dot(hfull, S, preferred_element_type=jnp.float32) * (1.0 / H)
    hb = hm + bias
    return jnp.maximum(hb, 0.0) + 0.01 * jnp.minimum(hb, 0.0)


def _tc2_body(n0_ref, n1_ref, d0_ref, d1_ref, b_ref, wl_ref, bl_ref,
              wr_ref, br_ref, att_ref, xl_ref, xr_ref):
    h1 = _combine_h(n0_ref[...], n1_ref[...], d0_ref[...], d1_ref[...],
                    b_ref[...])
    att_row = att_ref[...]
    xl = jnp.dot(h1, wl_ref[...],
                 preferred_element_type=jnp.float32) + bl_ref[...]
    xr = jnp.dot(h1, wr_ref[...],
                 preferred_element_type=jnp.float32) + br_ref[...]
    xl_ref[...] = _adot_table(xl, att_row)
    xr_ref[...] = _adot_table(xr, att_row)


def _tc2(num, den, bias1, Wl2p, bl2p, Wr2p, br2p, att_row):
    return pl.pallas_call(
        _tc2_body,
        grid=(TGRID,),
        in_specs=[
            pl.BlockSpec((BN, H * C), lambda i: (i, 0)),
            pl.BlockSpec((BN, H * C), lambda i: (TGRID + i, 0)),
            pl.BlockSpec((BN, H), lambda i: (i, 0)),
            pl.BlockSpec((BN, H), lambda i: (TGRID + i, 0)),
            pl.BlockSpec((1, C), lambda i: (0, 0)),
            pl.BlockSpec((C, H * C), lambda i: (0, 0)),
            pl.BlockSpec((1, H * C), lambda i: (0, 0)),
            pl.BlockSpec((C, H * C), lambda i: (0, 0)),
            pl.BlockSpec((1, H * C), lambda i: (0, 0)),
            pl.BlockSpec((1, H * C), lambda i: (0, 0)),
        ],
        out_specs=[
            pl.BlockSpec((BN, TW), lambda i: (i, 0)),
            pl.BlockSpec((BN, TW), lambda i: (i, 0)),
        ],
        out_shape=[
            jax.ShapeDtypeStruct((N, TW), jnp.float32),
            jax.ShapeDtypeStruct((N, TW), jnp.float32),
        ],
    )(num, num, den, den, bias1[None, :], Wl2p, bl2p[None, :], Wr2p,
      br2p[None, :], att_row[None, :])


def _tc3_body(n0_ref, n1_ref, d0_ref, d1_ref, b_ref, batch_ref, wc_ref,
              bc_ref, o_ref, pool_ref, cnt_ref):
    i = pl.program_id(0)
    h2 = _combine_h(n0_ref[...], n1_ref[...], d0_ref[...], d1_ref[...],
                    b_ref[...])
    b = batch_ref[0, 0, :]
    g = jax.lax.broadcasted_iota(jnp.int32, (NGRAPH, BN), 0)
    oh = (b[None, :] == g).astype(jnp.float32)
    pool = jnp.dot(oh, h2, preferred_element_type=jnp.float32)
    cnt = jnp.sum(oh, axis=1, keepdims=True)

    @pl.when(i == 0)
    def _():
        pool_ref[...] = jnp.zeros_like(pool_ref)
        cnt_ref[...] = jnp.zeros_like(cnt_ref)

    pool_ref[...] += pool
    cnt_ref[...] += cnt

    @pl.when(i == TGRID - 1)
    def _():
        pooled = pool_ref[...] / jnp.maximum(cnt_ref[...], 1.0)
        o_ref[...] = jnp.dot(pooled, wc_ref[...],
                             preferred_element_type=jnp.float32) + bc_ref[...]


def _tc3(num, den, bias2, batch3, Wc, bc):
    return pl.pallas_call(
        _tc3_body,
        grid=(TGRID,),
        in_specs=[
            pl.BlockSpec((BN, H * C), lambda i: (i, 0)),
            pl.BlockSpec((BN, H * C), lambda i: (TGRID + i, 0)),
            pl.BlockSpec((BN, H), lambda i: (i, 0)),
            pl.BlockSpec((BN, H), lambda i: (TGRID + i, 0)),
            pl.BlockSpec((1, C), lambda i: (0, 0)),
            pl.BlockSpec((1, 1, BN), lambda i: (i, 0, 0)),
            pl.BlockSpec((C, NCLASS), lambda i: (0, 0)),
            pl.BlockSpec((1, NCLASS), lambda i: (0, 0)),
        ],
        out_specs=pl.BlockSpec((NGRAPH, NCLASS), lambda i: (0, 0)),
        out_shape=jax.ShapeDtypeStruct((NGRAPH, NCLASS), jnp.float32),
        scratch_shapes=[
            pltpu.VMEM((NGRAPH, C), jnp.float32),
            pltpu.VMEM((NGRAPH, 1), jnp.float32),
        ],
    )(num, num, den, den, bias2[None, :], batch3, Wc, bc[None, :])


def _perm_w(Wl, bl):
    Wlp = Wl.reshape(-1, H, C).transpose(0, 2, 1).reshape(-1, H * C)
    blp = bl.reshape(H, C).T.reshape(-1)
    return Wlp, blp


def _prep_edges(edge_index):
    dst = edge_index[1]
    order = jnp.argsort(dst)
    srcs = edge_index[0][order]
    dsts = dst[order]
    h0 = jnp.searchsorted(dsts[:ES], jnp.arange(N + 1), side="left")
    h1 = jnp.searchsorted(dsts[ES:], jnp.arange(N + 1), side="left")
    rptr = jnp.stack([h0, h1]).astype(jnp.int32)
    rptr = jnp.concatenate(
        [rptr, jnp.full((NSC, RPTR_W - (N + 1)), ES, jnp.int32)], axis=1)
    srcs2 = jnp.concatenate(
        [srcs[:ES], jnp.zeros((PAD,), jnp.int32),
         srcs[ES:], jnp.zeros((PAD,), jnp.int32)])
    return srcs2.astype(jnp.int32), rptr.reshape(-1)


def kernel(x, edge_index, batch, Wl1, bl1, Wr1, br1, att1, bias1,
           Wl2, bl2, Wr2, br2, att2, bias2, Wc, bc):
    srcs, rptr = _prep_edges(edge_index)
    Wl1p, bl1p = _perm_w(Wl1, bl1)
    Wr1p, br1p = _perm_w(Wr1, br1)
    Wl2p, bl2p = _perm_w(Wl2, bl2)
    Wr2p, br2p = _perm_w(Wr2, br2)
    att1p = att1.T.reshape(-1)
    att2p = att2.T.reshape(-1)
    xl1, xr1 = _tc1(x, Wl1p, bl1p, Wr1p, br1p, att1p)
    num1, den1 = _sc_layer(xl1, xr1, 0.4 * att1p, srcs, rptr)
    xl2, xr2 = _tc2(num1, den1, bias1, Wl2p, bl2p, Wr2p, br2p, att2p)
    num2, den2 = _sc_layer(xl2, xr2, 0.4 * att2p, srcs, rptr)
    batch3 = batch.astype(jnp.int32).reshape(TGRID, 1, BN)
    return _tc3(num2, den2, bias2, batch3, Wc, bc)


# speedup vs baseline: 1.0509x; 1.0509x over previous
import functools

import jax
import jax.numpy as jnp
from jax import lax
from jax.experimental import pallas as pl
from jax.experimental.pallas import tpu as pltpu
from jax.experimental.pallas import tpu_sc as plsc

N = 10000
E = 320000
F_IN = 128
H = 16
C = 64
NCLASS = 10
NGRAPH = 64

NSC = 2
NSUB = 16
ES = E // NSC
PAD = 1056
DPT = 624
DG = 8
EC = 16
RPTR_W = N + 64
SBLK = 1040
TW = 1152
CLAMP = 60.0


def _sc_layer_body(xl_hbm, xr_hbm, att_hbm, srcs_hbm, rptr_hbm,
                   num_hbm, den_hbm,
                   attv, rptrv, xrg, xjb0, xjb1, outg, deng, sblk,
                   gsem0, gsem1,
                   interp=False):
    if interp:
        s, w = pl.program_id(0), pl.program_id(1)
    else:
        s = lax.axis_index("c")
        w = lax.axis_index("s")

    pltpu.sync_copy(att_hbm, attv)

    tile_d0 = w * DPT
    ngrp = jnp.where(w == NSUB - 1, (N - DPT * (NSUB - 1)) // DG, DPT // DG)
    pltpu.sync_copy(rptr_hbm.at[pl.ds(s * RPTR_W + tile_d0, 656)], rptrv)
    src_base = s * (ES + PAD)

    def group_body(g, _):
        d0 = tile_d0 + g * DG
        pltpu.sync_copy(xr_hbm.at[pl.ds(d0, DG)], xrg)

        zero = jnp.zeros((16,), jnp.float32)

        def zero_dl(dl, _):
            deng[dl, :] = zero

            def zero_c(c, _):
                outg[dl, pl.ds(c * H, H)] = zero
                return 0
            lax.fori_loop(0, C, zero_c, 0)
            return 0
        lax.fori_loop(0, DG, zero_dl, 0)

        def seg_at(dl):
            dlc = jnp.minimum(dl, DG)
            rv = rptrv[pl.ds(g * DG + dlc, 16)]
            ss = rv[0]
            ln = jnp.where(dl < DG, rv[1] - ss, 0)
            return ss, ln

        def valid(st):
            dl, k, ss, ln = st
            return (dl < DG) & (k * EC < ln)

        def advance(st):
            dl, k, ss, ln = st
            more = (k + 1) * EC < ln
            dl2 = jnp.where(more, dl, jnp.minimum(dl + 1, DG))
            k2 = jnp.where(more, k + 1, 0)
            ss2, ln2 = seg_at(dl2)
            ss2 = jnp.where(more, ss, ss2)
            ln2 = jnp.where(more, ln, ln2)
            return (dl2, k2, ss2, ln2)

        def issue(st, blk_lo, buf, sem):
            dl, k, ss, ln = st
            v = valid(st)
            cs = ss + k * EC
            need = v & ((cs < blk_lo) | (cs + EC > blk_lo + SBLK))
            cs8 = (cs // 8) * 8
            blk_lo2 = jnp.where(need, cs8, blk_lo)

            @pl.when(need)
            def _():
                pltpu.sync_copy(srcs_hbm.at[pl.ds(src_base + cs8, SBLK)],
                                sblk)

            @pl.when(v)
            def _():
                srcv = sblk[pl.ds(cs - blk_lo2, EC)]
                pltpu.async_copy(xl_hbm.at[srcv], buf, sem)
            return blk_lo2

        def compute(st, buf, sem):
            v = valid(st)

            @pl.when(v)
            def _():
                pltpu.make_async_copy(xl_hbm.at[pl.ds(0, EC)], buf, sem).wait()

            if True:
                dl, k, ss, ln = st
                dl = jnp.minimum(dl, DG - 1)
                accs = [jnp.zeros((16,), jnp.float32) for _ in range(EC)]

                def chan_body(c, accs):
                    ch = pl.ds(c * H, H)
                    att_c = attv[ch]
                    xi_c = xrg[dl, ch]
                    out = []
                    for e in range(EC):
                        u = xi_c + buf[e, ch]
                        out.append(accs[e] + att_c * jnp.abs(u))
                    return out
                accs = lax.fori_loop(0, C, chan_body, accs)

                arv = xrg[dl, pl.ds(C * H, H)]
                rem = ln - k * EC
                ps = []
                for e in range(EC):
                    l_e = accs[e] + (buf[e, pl.ds(C * H, H)] + arv)
                    l_e = jnp.minimum(jnp.maximum(l_e, -CLAMP), CLAMP)
                    p_e = jnp.exp(l_e)
                    ps.append(jnp.where(rem > e, p_e, jnp.zeros_like(p_e)))
                dsum = ps[0]
                for e in range(1, EC):
                    dsum = dsum + ps[e]
                deng[dl, :] = deng[dl, :] + dsum

                def agg_body(c, _):
                    ch = pl.ds(c * H, H)
                    o = outg[dl, ch]
                    for e in range(EC):
                        o = o + ps[e] * buf[e, ch]
                    outg[dl, ch] = o
                    return 0
                lax.fori_loop(0, C, agg_body, 0)

        def trip_body(dl, t):
            _, ln = seg_at(dl)
            return t + jnp.maximum((ln + EC - 1) // EC, 1)

        trip = lax.fori_loop(0, DG, trip_body, jnp.int32(0))

        bufs = (xjb0, xjb1)
        sems = (gsem0, gsem1)
        ss0, ln0 = seg_at(jnp.int32(0))
        st = (jnp.int32(0), jnp.int32(0), ss0, ln0)
        blk_lo = jnp.int32(-2**30)
        sts = []
        for r in range(2):
            sts.append(st)
            blk_lo = issue(st, blk_lo, bufs[r], sems[r])
            st = advance(st)

        def body(q, carry):
            st0, st1, stn, blk_lo = carry
            sts = [st0, st1]
            for r in range(2):
                compute(sts[r], bufs[r], sems[r])
                blk_lo = issue(stn, blk_lo, bufs[r], sems[r])
                sts[r] = stn
                stn = advance(stn)
            return (sts[0], sts[1], stn, blk_lo)

        lax.fori_loop(0, (trip + 1) // 2, body,
                      (sts[0], sts[1], st, blk_lo))

        pltpu.sync_copy(outg, num_hbm.at[pl.ds(s * N + d0, DG)])
        pltpu.sync_copy(deng, den_hbm.at[pl.ds(s * N + d0, DG)])
        return 0

    lax.fori_loop(0, ngrp, group_body, 0)


@functools.partial(jax.jit, static_argnames=("interpret",))
def _sc_layer(xl, xr, attp, srcs, rptr, interpret=False):
    mesh = plsc.VectorSubcoreMesh(core_axis_name="c", subcore_axis_name="s",
                                  num_cores=NSC, num_subcores=NSUB)
    f = pl.kernel(
        functools.partial(_sc_layer_body, interp=interpret),
        out_type=[
            jax.ShapeDtypeStruct((NSC * N, C * H), jnp.float32),
            jax.ShapeDtypeStruct((NSC * N, H), jnp.float32),
        ],
        mesh=mesh,
        scratch_types=[
            pltpu.VMEM((C * H,), jnp.float32),
            pltpu.VMEM((656,), jnp.int32),
            pltpu.VMEM((DG, TW), jnp.float32),
            pltpu.VMEM((EC, TW), jnp.float32),
            pltpu.VMEM((EC, TW), jnp.float32),
            pltpu.VMEM((DG, C * H), jnp.float32),
            pltpu.VMEM((DG, H), jnp.float32),
            pltpu.VMEM((SBLK,), jnp.int32),
            pltpu.SemaphoreType.DMA,
            pltpu.SemaphoreType.DMA,
        ],
        interpret=interpret,
    )
    return f(xl, xr, attp, srcs, rptr)


BN = 400
TGRID = N // BN


def _adot_table(v, att_row):
    rows = jax.lax.broadcasted_iota(jnp.int32, (H * C, H), 0) % H
    colh = jax.lax.broadcasted_iota(jnp.int32, (H * C, H), 1)
    T2 = (rows == colh).astype(jnp.float32)
    a = jnp.dot(v * att_row, T2, preferred_element_type=jnp.float32) * 0.6
    pad = jnp.zeros((v.shape[0], TW - H * C - H), jnp.float32)
    return jnp.concatenate([v, a, pad], axis=1)


def _tc1_body(x_ref, wl_ref, bl_ref, wr_ref, br_ref, att_ref,
              xl_ref, xr_ref):
    x = x_ref[...]
    att_row = att_ref[...]
    xl = jnp.dot(x, wl_ref[...],
                 preferred_element_type=jnp.float32) + bl_ref[...]
    xr = jnp.dot(x, wr_ref[...],
                 preferred_element_type=jnp.float32) + br_ref[...]
    xl_ref[...] = _adot_table(xl, att_row)
    xr_ref[...] = _adot_table(xr, att_row)


def _tc1(x, Wlp, blp, Wrp, brp, att_row):
    f_in = x.shape[1]
    return pl.pallas_call(
        _tc1_body,
        grid=(TGRID,),
        in_specs=[
            pl.BlockSpec((BN, f_in), lambda i: (i, 0)),
            pl.BlockSpec((f_in, H * C), lambda i: (0, 0)),
            pl.BlockSpec((1, H * C), lambda i: (0, 0)),
            pl.BlockSpec((f_in, H * C), lambda i: (0, 0)),
            pl.BlockSpec((1, H * C), lambda i: (0, 0)),
            pl.BlockSpec((1, H * C), lambda i: (0, 0)),
        ],
        out_specs=[
            pl.BlockSpec((BN, TW), lambda i: (i, 0)),
            pl.BlockSpec((BN, TW), lambda i: (i, 0)),
        ],
        out_shape=[
            jax.ShapeDtypeStruct((N, TW), jnp.float32),
            jax.ShapeDtypeStruct((N, TW), jnp.float32),
        ],
    )(x, Wlp, blp[None, :], Wrp, brp[None, :], att_row[None, :])


def _combine_h(n0, n1, d0, d1, bias):
    dsum = d0 + d1 + 1e-16
    col = jax.lax.broadcasted_iota(jnp.int32, (H, H * C), 1) % H
    rowh = jax.lax.broadcasted_iota(jnp.int32, (H, H * C), 0)
    T = (col == rowh).astype(jnp.float32)
    den_b = jnp.dot(dsum, T, preferred_element_type=jnp.float32)
    hfull = (n0 + n1) / den_b
    rows = jax.lax.broadcasted_iota(jnp.int32, (H * C, C), 0) // H
    colc = jax.lax.broadcasted_iota(jnp.int32, (H * C, C), 1)
    S = (rows == colc).astype(jnp.float32)
    hm = jnp.dot(hfull, S, preferred_element_type=jnp.float32) * (1.0 / H)
    hb = hm + bias
    return jnp.maximum(hb, 0.0) + 0.01 * jnp.minimum(hb, 0.0)


def _tc2_body(n0_ref, n1_ref, d0_ref, d1_ref, b_ref, wl_ref, bl_ref,
              wr_ref, br_ref, att_ref, xl_ref, xr_ref):
    h1 = _combine_h(n0_ref[...], n1_ref[...], d0_ref[...], d1_ref[...],
                    b_ref[...])
    att_row = att_ref[...]
    xl = jnp.dot(h1, wl_ref[...],
                 preferred_element_type=jnp.float32) + bl_ref[...]
    xr = jnp.dot(h1, wr_ref[...],
                 preferred_element_type=jnp.float32) + br_ref[...]
    xl_ref[...] = _adot_table(xl, att_row)
    xr_ref[...] = _adot_table(xr, att_row)


def _tc2(num, den, bias1, Wl2p, bl2p, Wr2p, br2p, att_row):
    return pl.pallas_call(
        _tc2_body,
        grid=(TGRID,),
        in_specs=[
            pl.BlockSpec((BN, H * C), lambda i: (i, 0)),
            pl.BlockSpec((BN, H * C), lambda i: (TGRID + i, 0)),
            pl.BlockSpec((BN, H), lambda i: (i, 0)),
            pl.BlockSpec((BN, H), lambda i: (TGRID + i, 0)),
            pl.BlockSpec((1, C), lambda i: (0, 0)),
            pl.BlockSpec((C, H * C), lambda i: (0, 0)),
            pl.BlockSpec((1, H * C), lambda i: (0, 0)),
            pl.BlockSpec((C, H * C), lambda i: (0, 0)),
            pl.BlockSpec((1, H * C), lambda i: (0, 0)),
            pl.BlockSpec((1, H * C), lambda i: (0, 0)),
        ],
        out_specs=[
            pl.BlockSpec((BN, TW), lambda i: (i, 0)),
            pl.BlockSpec((BN, TW), lambda i: (i, 0)),
        ],
        out_shape=[
            jax.ShapeDtypeStruct((N, TW), jnp.float32),
            jax.ShapeDtypeStruct((N, TW), jnp.float32),
        ],
    )(num, num, den, den, bias1[None, :], Wl2p, bl2p[None, :], Wr2p,
      br2p[None, :], att_row[None, :])


def _tc3_body(n0_ref, n1_ref, d0_ref, d1_ref, b_ref, batch_ref, wc_ref,
              bc_ref, o_ref, pool_ref, cnt_ref):
    i = pl.program_id(0)
    h2 = _combine_h(n0_ref[...], n1_ref[...], d0_ref[...], d1_ref[...],
                    b_ref[...])
    b = batch_ref[0, 0, :]
    g = jax.lax.broadcasted_iota(jnp.int32, (NGRAPH, BN), 0)
    oh = (b[None, :] == g).astype(jnp.float32)
    pool = jnp.dot(oh, h2, preferred_element_type=jnp.float32)
    cnt = jnp.sum(oh, axis=1, keepdims=True)

    @pl.when(i == 0)
    def _():
        pool_ref[...] = jnp.zeros_like(pool_ref)
        cnt_ref[...] = jnp.zeros_like(cnt_ref)

    pool_ref[...] += pool
    cnt_ref[...] += cnt

    @pl.when(i == TGRID - 1)
    def _():
        pooled = pool_ref[...] / jnp.maximum(cnt_ref[...], 1.0)
        o_ref[...] = jnp.dot(pooled, wc_ref[...],
                             preferred_element_type=jnp.float32) + bc_ref[...]


def _tc3(num, den, bias2, batch3, Wc, bc):
    return pl.pallas_call(
        _tc3_body,
        grid=(TGRID,),
        in_specs=[
            pl.BlockSpec((BN, H * C), lambda i: (i, 0)),
            pl.BlockSpec((BN, H * C), lambda i: (TGRID + i, 0)),
            pl.BlockSpec((BN, H), lambda i: (i, 0)),
            pl.BlockSpec((BN, H), lambda i: (TGRID + i, 0)),
            pl.BlockSpec((1, C), lambda i: (0, 0)),
            pl.BlockSpec((1, 1, BN), lambda i: (i, 0, 0)),
            pl.BlockSpec((C, NCLASS), lambda i: (0, 0)),
            pl.BlockSpec((1, NCLASS), lambda i: (0, 0)),
        ],
        out_specs=pl.BlockSpec((NGRAPH, NCLASS), lambda i: (0, 0)),
        out_shape=jax.ShapeDtypeStruct((NGRAPH, NCLASS), jnp.float32),
        scratch_shapes=[
            pltpu.VMEM((NGRAPH, C), jnp.float32),
            pltpu.VMEM((NGRAPH, 1), jnp.float32),
        ],
    )(num, num, den, den, bias2[None, :], batch3, Wc, bc[None, :])


def _perm_w(Wl, bl):
    Wlp = Wl.reshape(-1, H, C).transpose(0, 2, 1).reshape(-1, H * C)
    blp = bl.reshape(H, C).T.reshape(-1)
    return Wlp, blp


def _prep_edges(edge_index):
    keys = jnp.sort(edge_index[1] * 16384 + edge_index[0])
    srcs = keys & 16383
    bounds = jnp.arange(N + 1) * 16384
    h0 = jnp.searchsorted(keys[:ES], bounds, side="left")
    h1 = jnp.searchsorted(keys[ES:], bounds, side="left")
    rptr = jnp.stack([h0, h1]).astype(jnp.int32)
    rptr = jnp.concatenate(
        [rptr, jnp.full((NSC, RPTR_W - (N + 1)), ES, jnp.int32)], axis=1)
    srcs2 = jnp.concatenate(
        [srcs[:ES], jnp.zeros((PAD,), jnp.int32),
         srcs[ES:], jnp.zeros((PAD,), jnp.int32)])
    return srcs2.astype(jnp.int32), rptr.reshape(-1)


def kernel(x, edge_index, batch, Wl1, bl1, Wr1, br1, att1, bias1,
           Wl2, bl2, Wr2, br2, att2, bias2, Wc, bc):
    srcs, rptr = _prep_edges(edge_index)
    Wl1p, bl1p = _perm_w(Wl1, bl1)
    Wr1p, br1p = _perm_w(Wr1, br1)
    Wl2p, bl2p = _perm_w(Wl2, bl2)
    Wr2p, br2p = _perm_w(Wr2, br2)
    att1p = att1.T.reshape(-1)
    att2p = att2.T.reshape(-1)
    xl1, xr1 = _tc1(x, Wl1p, bl1p, Wr1p, br1p, att1p)
    num1, den1 = _sc_layer(xl1, xr1, 0.4 * att1p, srcs, rptr)
    xl2, xr2 = _tc2(num1, den1, bias1, Wl2p, bl2p, Wr2p, br2p, att2p)
    num2, den2 = _sc_layer(xl2, xr2, 0.4 * att2p, srcs, rptr)
    batch3 = batch.astype(jnp.int32).reshape(TGRID, 1, BN)
    return _tc3(num2, den2, bias2, batch3, Wc, bc)


# speedup vs baseline: 1.0796x; 1.0273x over previous
import functools

import jax
import jax.numpy as jnp
from jax import lax
from jax.experimental import pallas as pl
from jax.experimental.pallas import tpu as pltpu
from jax.experimental.pallas import tpu_sc as plsc

N = 10000
E = 320000
F_IN = 128
H = 16
C = 64
NCLASS = 10
NGRAPH = 64

NSC = 2
NSUB = 16
ES = E // NSC
PAD = 1056
DPT = 624
DG = 16
EC = 16
RPTR_W = N + 64
SBLK = 1040
TW = 1152
CLAMP = 60.0


def _sc_layer_body(xl_hbm, xr_hbm, att_hbm, srcs_hbm, rptr_hbm,
                   num_hbm, den_hbm,
                   attv, rptrv, xrg, xjb0, xjb1, outg, deng, sblk,
                   gsem0, gsem1,
                   interp=False):
    if interp:
        s, w = pl.program_id(0), pl.program_id(1)
    else:
        s = lax.axis_index("c")
        w = lax.axis_index("s")

    pltpu.sync_copy(att_hbm, attv)

    tile_d0 = w * DPT
    ngrp = jnp.where(w == NSUB - 1, (N - DPT * (NSUB - 1)) // DG, DPT // DG)
    pltpu.sync_copy(rptr_hbm.at[pl.ds(s * RPTR_W + tile_d0, 656)], rptrv)
    src_base = s * (ES + PAD)

    def group_body(g, _):
        d0 = tile_d0 + g * DG
        pltpu.sync_copy(xr_hbm.at[pl.ds(d0, DG)], xrg)

        zero = jnp.zeros((16,), jnp.float32)

        def zero_dl(dl, _):
            deng[dl, :] = zero

            def zero_c(c, _):
                outg[dl, pl.ds(c * H, H)] = zero
                return 0
            lax.fori_loop(0, C, zero_c, 0)
            return 0
        lax.fori_loop(0, DG, zero_dl, 0)

        def seg_at(dl):
            dlc = jnp.minimum(dl, DG)
            rv = rptrv[pl.ds(g * DG + dlc, 16)]
            ss = rv[0]
            ln = jnp.where(dl < DG, rv[1] - ss, 0)
            return ss, ln

        def valid(st):
            dl, k, ss, ln = st
            return (dl < DG) & (k * EC < ln)

        def advance(st):
            dl, k, ss, ln = st
            more = (k + 1) * EC < ln
            dl2 = jnp.where(more, dl, jnp.minimum(dl + 1, DG))
            k2 = jnp.where(more, k + 1, 0)
            ss2, ln2 = seg_at(dl2)
            ss2 = jnp.where(more, ss, ss2)
            ln2 = jnp.where(more, ln, ln2)
            return (dl2, k2, ss2, ln2)

        def issue(st, blk_lo, buf, sem):
            dl, k, ss, ln = st
            v = valid(st)
            cs = ss + k * EC
            need = v & ((cs < blk_lo) | (cs + EC > blk_lo + SBLK))
            cs8 = (cs // 8) * 8
            blk_lo2 = jnp.where(need, cs8, blk_lo)

            @pl.when(need)
            def _():
                pltpu.sync_copy(srcs_hbm.at[pl.ds(src_base + cs8, SBLK)],
                                sblk)

            @pl.when(v)
            def _():
                srcv = sblk[pl.ds(cs - blk_lo2, EC)]
                pltpu.async_copy(xl_hbm.at[srcv], buf, sem)
            return blk_lo2

        def compute(st, buf, sem):
            v = valid(st)

            @pl.when(v)
            def _():
                pltpu.make_async_copy(xl_hbm.at[pl.ds(0, EC)], buf, sem).wait()

            if True:
                dl, k, ss, ln = st
                dl = jnp.minimum(dl, DG - 1)
                accs = [jnp.zeros((16,), jnp.float32) for _ in range(EC)]

                def chan_body(c, accs):
                    ch = pl.ds(c * H, H)
                    att_c = attv[ch]
                    xi_c = xrg[dl, ch]
                    out = []
                    for e in range(EC):
                        u = xi_c + buf[e, ch]
                        out.append(accs[e] + att_c * jnp.abs(u))
                    return out
                accs = lax.fori_loop(0, C, chan_body, accs)

                arv = xrg[dl, pl.ds(C * H, H)]
                rem = ln - k * EC
                ps = []
                for e in range(EC):
                    l_e = accs[e] + (buf[e, pl.ds(C * H, H)] + arv)
                    l_e = jnp.minimum(jnp.maximum(l_e, -CLAMP), CLAMP)
                    p_e = jnp.exp(l_e)
                    ps.append(jnp.where(rem > e, p_e, jnp.zeros_like(p_e)))
                dsum = ps[0]
                for e in range(1, EC):
                    dsum = dsum + ps[e]
                deng[dl, :] = deng[dl, :] + dsum

                def agg_body(c, _):
                    ch = pl.ds(c * H, H)
                    o = outg[dl, ch]
                    for e in range(EC):
                        o = o + ps[e] * buf[e, ch]
                    outg[dl, ch] = o
                    return 0
                lax.fori_loop(0, C, agg_body, 0)

        def trip_body(dl, t):
            _, ln = seg_at(dl)
            return t + jnp.maximum((ln + EC - 1) // EC, 1)

        trip = lax.fori_loop(0, DG, trip_body, jnp.int32(0))

        bufs = (xjb0, xjb1)
        sems = (gsem0, gsem1)
        ss0, ln0 = seg_at(jnp.int32(0))
        st = (jnp.int32(0), jnp.int32(0), ss0, ln0)
        blk_lo = jnp.int32(-2**30)
        sts = []
        for r in range(2):
            sts.append(st)
            blk_lo = issue(st, blk_lo, bufs[r], sems[r])
            st = advance(st)

        def body(q, carry):
            st0, st1, stn, blk_lo = carry
            sts = [st0, st1]
            for r in range(2):
                compute(sts[r], bufs[r], sems[r])
                blk_lo = issue(stn, blk_lo, bufs[r], sems[r])
                sts[r] = stn
                stn = advance(stn)
            return (sts[0], sts[1], stn, blk_lo)

        lax.fori_loop(0, (trip + 1) // 2, body,
                      (sts[0], sts[1], st, blk_lo))

        pltpu.sync_copy(outg, num_hbm.at[pl.ds(s * N + d0, DG)])
        pltpu.sync_copy(deng, den_hbm.at[pl.ds(s * N + d0, DG)])
        return 0

    lax.fori_loop(0, ngrp, group_body, 0)


@functools.partial(jax.jit, static_argnames=("interpret",))
def _sc_layer(xl, xr, attp, srcs, rptr, interpret=False):
    mesh = plsc.VectorSubcoreMesh(core_axis_name="c", subcore_axis_name="s",
                                  num_cores=NSC, num_subcores=NSUB)
    f = pl.kernel(
        functools.partial(_sc_layer_body, interp=interpret),
        out_type=[
            jax.ShapeDtypeStruct((NSC * N, C * H), jnp.float32),
            jax.ShapeDtypeStruct((NSC * N, H), jnp.float32),
        ],
        mesh=mesh,
        scratch_types=[
            pltpu.VMEM((C * H,), jnp.float32),
            pltpu.VMEM((656,), jnp.int32),
            pltpu.VMEM((DG, TW), jnp.float32),
            pltpu.VMEM((EC, TW), jnp.float32),
            pltpu.VMEM((EC, TW), jnp.float32),
            pltpu.VMEM((DG, C * H), jnp.float32),
            pltpu.VMEM((DG, H), jnp.float32),
            pltpu.VMEM((SBLK,), jnp.int32),
            pltpu.SemaphoreType.DMA,
            pltpu.SemaphoreType.DMA,
        ],
        interpret=interpret,
    )
    return f(xl, xr, attp, srcs, rptr)


BN = 400
TGRID = N // BN


def _adot_table(v, att_row):
    rows = jax.lax.broadcasted_iota(jnp.int32, (H * C, H), 0) % H
    colh = jax.lax.broadcasted_iota(jnp.int32, (H * C, H), 1)
    T2 = (rows == colh).astype(jnp.float32)
    a = jnp.dot(v * att_row, T2, preferred_element_type=jnp.float32) * 0.6
    pad = jnp.zeros((v.shape[0], TW - H * C - H), jnp.float32)
    return jnp.concatenate([v, a, pad], axis=1)


def _tc1_body(x_ref, wl_ref, bl_ref, wr_ref, br_ref, att_ref,
              xl_ref, xr_ref):
    x = x_ref[...]
    att_row = att_ref[...]
    xl = jnp.dot(x, wl_ref[...],
                 preferred_element_type=jnp.float32) + bl_ref[...]
    xr = jnp.dot(x, wr_ref[...],
                 preferred_element_type=jnp.float32) + br_ref[...]
    xl_ref[...] = _adot_table(xl, att_row)
    xr_ref[...] = _adot_table(xr, att_row)


def _tc1(x, Wlp, blp, Wrp, brp, att_row):
    f_in = x.shape[1]
    return pl.pallas_call(
        _tc1_body,
        grid=(TGRID,),
        in_specs=[
            pl.BlockSpec((BN, f_in), lambda i: (i, 0)),
            pl.BlockSpec((f_in, H * C), lambda i: (0, 0)),
            pl.BlockSpec((1, H * C), lambda i: (0, 0)),
            pl.BlockSpec((f_in, H * C), lambda i: (0, 0)),
            pl.BlockSpec((1, H * C), lambda i: (0, 0)),
            pl.BlockSpec((1, H * C), lambda i: (0, 0)),
        ],
        out_specs=[
            pl.BlockSpec((BN, TW), lambda i: (i, 0)),
            pl.BlockSpec((BN, TW), lambda i: (i, 0)),
        ],
        out_shape=[
            jax.ShapeDtypeStruct((N, TW), jnp.float32),
            jax.ShapeDtypeStruct((N, TW), jnp.float32),
        ],
    )(x, Wlp, blp[None, :], Wrp, brp[None, :], att_row[None, :])


def _combine_h(n0, n1, d0, d1, bias):
    dsum = d0 + d1 + 1e-16
    col = jax.lax.broadcasted_iota(jnp.int32, (H, H * C), 1) % H
    rowh = jax.lax.broadcasted_iota(jnp.int32, (H, H * C), 0)
    T = (col == rowh).astype(jnp.float32)
    den_b = jnp.dot(dsum, T, preferred_element_type=jnp.float32)
    hfull = (n0 + n1) / den_b
    rows = jax.lax.broadcasted_iota(jnp.int32, (H * C, C), 0) // H
    colc = jax.lax.broadcasted_iota(jnp.int32, (H * C, C), 1)
    S = (rows == colc).astype(jnp.float32)
    hm = jnp.dot(hfull, S, preferred_element_type=jnp.float32) * (1.0 / H)
    hb = hm + bias
    return jnp.maximum(hb, 0.0) + 0.01 * jnp.minimum(hb, 0.0)


def _tc2_body(n0_ref, n1_ref, d0_ref, d1_ref, b_ref, wl_ref, bl_ref,
              wr_ref, br_ref, att_ref, xl_ref, xr_ref):
    h1 = _combine_h(n0_ref[...], n1_ref[...], d0_ref[...], d1_ref[...],
                    b_ref[...])
    att_row = att_ref[...]
    xl = jnp.dot(h1, wl_ref[...],
                 preferred_element_type=jnp.float32) + bl_ref[...]
    xr = jnp.dot(h1, wr_ref[...],
                 preferred_element_type=jnp.float32) + br_ref[...]
    xl_ref[...] = _adot_table(xl, att_row)
    xr_ref[...] = _adot_table(xr, att_row)


def _tc2(num, den, bias1, Wl2p, bl2p, Wr2p, br2p, att_row):
    return pl.pallas_call(
        _tc2_body,
        grid=(TGRID,),
        in_specs=[
            pl.BlockSpec((BN, H * C), lambda i: (i, 0)),
            pl.BlockSpec((BN, H * C), lambda i: (TGRID + i, 0)),
            pl.BlockSpec((BN, H), lambda i: (i, 0)),
            pl.BlockSpec((BN, H), lambda i: (TGRID + i, 0)),
            pl.BlockSpec((1, C), lambda i: (0, 0)),
            pl.BlockSpec((C, H * C), lambda i: (0, 0)),
            pl.BlockSpec((1, H * C), lambda i: (0, 0)),
            pl.BlockSpec((C, H * C), lambda i: (0, 0)),
            pl.BlockSpec((1, H * C), lambda i: (0, 0)),
            pl.BlockSpec((1, H * C), lambda i: (0, 0)),
        ],
        out_specs=[
            pl.BlockSpec((BN, TW), lambda i: (i, 0)),
            pl.BlockSpec((BN, TW), lambda i: (i, 0)),
        ],
        out_shape=[
            jax.ShapeDtypeStruct((N, TW), jnp.float32),
            jax.ShapeDtypeStruct((N, TW), jnp.float32),
        ],
    )(num, num, den, den, bias1[None, :], Wl2p, bl2p[None, :], Wr2p,
      br2p[None, :], att_row[None, :])


def _tc3_body(n0_ref, n1_ref, d0_ref, d1_ref, b_ref, batch_ref, wc_ref,
              bc_ref, o_ref, pool_ref, cnt_ref):
    i = pl.program_id(0)
    h2 = _combine_h(n0_ref[...], n1_ref[...], d0_ref[...], d1_ref[...],
                    b_ref[...])
    b = batch_ref[0, 0, :]
    g = jax.lax.broadcasted_iota(jnp.int32, (NGRAPH, BN), 0)
    oh = (b[None, :] == g).astype(jnp.float32)
    pool = jnp.dot(oh, h2, preferred_element_type=jnp.float32)
    cnt = jnp.sum(oh, axis=1, keepdims=True)

    @pl.when(i == 0)
    def _():
        pool_ref[...] = jnp.zeros_like(pool_ref)
        cnt_ref[...] = jnp.zeros_like(cnt_ref)

    pool_ref[...] += pool
    cnt_ref[...] += cnt

    @pl.when(i == TGRID - 1)
    def _():
        pooled = pool_ref[...] / jnp.maximum(cnt_ref[...], 1.0)
        o_ref[...] = jnp.dot(pooled, wc_ref[...],
                             preferred_element_type=jnp.float32) + bc_ref[...]


def _tc3(num, den, bias2, batch3, Wc, bc):
    return pl.pallas_call(
        _tc3_body,
        grid=(TGRID,),
        in_specs=[
            pl.BlockSpec((BN, H * C), lambda i: (i, 0)),
            pl.BlockSpec((BN, H * C), lambda i: (TGRID + i, 0)),
            pl.BlockSpec((BN, H), lambda i: (i, 0)),
            pl.BlockSpec((BN, H), lambda i: (TGRID + i, 0)),
            pl.BlockSpec((1, C), lambda i: (0, 0)),
            pl.BlockSpec((1, 1, BN), lambda i: (i, 0, 0)),
            pl.BlockSpec((C, NCLASS), lambda i: (0, 0)),
            pl.BlockSpec((1, NCLASS), lambda i: (0, 0)),
        ],
        out_specs=pl.BlockSpec((NGRAPH, NCLASS), lambda i: (0, 0)),
        out_shape=jax.ShapeDtypeStruct((NGRAPH, NCLASS), jnp.float32),
        scratch_shapes=[
            pltpu.VMEM((NGRAPH, C), jnp.float32),
            pltpu.VMEM((NGRAPH, 1), jnp.float32),
        ],
    )(num, num, den, den, bias2[None, :], batch3, Wc, bc[None, :])


def _perm_w(Wl, bl):
    Wlp = Wl.reshape(-1, H, C).transpose(0, 2, 1).reshape(-1, H * C)
    blp = bl.reshape(H, C).T.reshape(-1)
    return Wlp, blp


def _prep_edges(edge_index):
    keys = jnp.sort(edge_index[1] * 16384 + edge_index[0])
    srcs = keys & 16383
    bounds = jnp.arange(N + 1) * 16384
    h0 = jnp.searchsorted(keys[:ES], bounds, side="left")
    h1 = jnp.searchsorted(keys[ES:], bounds, side="left")
    rptr = jnp.stack([h0, h1]).astype(jnp.int32)
    rptr = jnp.concatenate(
        [rptr, jnp.full((NSC, RPTR_W - (N + 1)), ES, jnp.int32)], axis=1)
    srcs2 = jnp.concatenate(
        [srcs[:ES], jnp.zeros((PAD,), jnp.int32),
         srcs[ES:], jnp.zeros((PAD,), jnp.int32)])
    return srcs2.astype(jnp.int32), rptr.reshape(-1)


def kernel(x, edge_index, batch, Wl1, bl1, Wr1, br1, att1, bias1,
           Wl2, bl2, Wr2, br2, att2, bias2, Wc, bc):
    srcs, rptr = _prep_edges(edge_index)
    Wl1p, bl1p = _perm_w(Wl1, bl1)
    Wr1p, br1p = _perm_w(Wr1, br1)
    Wl2p, bl2p = _perm_w(Wl2, bl2)
    Wr2p, br2p = _perm_w(Wr2, br2)
    att1p = att1.T.reshape(-1)
    att2p = att2.T.reshape(-1)
    xl1, xr1 = _tc1(x, Wl1p, bl1p, Wr1p, br1p, att1p)
    num1, den1 = _sc_layer(xl1, xr1, 0.4 * att1p, srcs, rptr)
    xl2, xr2 = _tc2(num1, den1, bias1, Wl2p, bl2p, Wr2p, br2p, att2p)
    num2, den2 = _sc_layer(xl2, xr2, 0.4 * att2p, srcs, rptr)
    batch3 = batch.astype(jnp.int32).reshape(TGRID, 1, BN)
    return _tc3(num2, den2, bias2, batch3, Wc, bc)


# speedup vs baseline: 1.0796x; 1.0000x over previous
import functools

import jax
import jax.numpy as jnp
from jax import lax
from jax.experimental import pallas as pl
from jax.experimental.pallas import tpu as pltpu
from jax.experimental.pallas import tpu_sc as plsc

N = 10000
E = 320000
F_IN = 128
H = 16
C = 64
NCLASS = 10
NGRAPH = 64

NSC = 2
NSUB = 16
ES = E // NSC
PAD = 1056
DPT = 624
DG = 16
EC = 16
RPTR_W = N + 64
SBLK = 1040
TW = 1152
CLAMP = 60.0


def _sc_layer_body(xl_hbm, xr_hbm, att_hbm, srcs_hbm, rptr_hbm,
                   num_hbm, den_hbm,
                   attv, rptrv, xrg, xjb0, xjb1, outg, deng, sblk,
                   gsem0, gsem1):
    s = lax.axis_index("c")
    w = lax.axis_index("s")

    pltpu.sync_copy(att_hbm, attv)

    tile_d0 = w * DPT
    ngrp = jnp.where(w == NSUB - 1, (N - DPT * (NSUB - 1)) // DG, DPT // DG)
    pltpu.sync_copy(rptr_hbm.at[pl.ds(s * RPTR_W + tile_d0, 656)], rptrv)
    src_base = s * (ES + PAD)

    def group_body(g, _):
        d0 = tile_d0 + g * DG
        pltpu.sync_copy(xr_hbm.at[pl.ds(d0, DG)], xrg)

        zero = jnp.zeros((16,), jnp.float32)

        def zero_dl(dl, _):
            deng[dl, :] = zero

            def zero_c(c, _):
                outg[dl, pl.ds(c * H, H)] = zero
                return 0
            lax.fori_loop(0, C, zero_c, 0)
            return 0
        lax.fori_loop(0, DG, zero_dl, 0)

        def seg_at(dl):
            dlc = jnp.minimum(dl, DG)
            rv = rptrv[pl.ds(g * DG + dlc, 16)]
            ss = rv[0]
            ln = jnp.where(dl < DG, rv[1] - ss, 0)
            return ss, ln

        def valid(st):
            dl, k, ss, ln = st
            return (dl < DG) & (k * EC < ln)

        def advance(st):
            dl, k, ss, ln = st
            more = (k + 1) * EC < ln
            dl2 = jnp.where(more, dl, jnp.minimum(dl + 1, DG))
            k2 = jnp.where(more, k + 1, 0)
            ss2, ln2 = seg_at(dl2)
            ss2 = jnp.where(more, ss, ss2)
            ln2 = jnp.where(more, ln, ln2)
            return (dl2, k2, ss2, ln2)

        def issue(st, blk_lo, buf, sem):
            dl, k, ss, ln = st
            v = valid(st)
            cs = ss + k * EC
            need = v & ((cs < blk_lo) | (cs + EC > blk_lo + SBLK))
            cs8 = (cs // 8) * 8
            blk_lo2 = jnp.where(need, cs8, blk_lo)

            @pl.when(need)
            def _():
                pltpu.sync_copy(srcs_hbm.at[pl.ds(src_base + cs8, SBLK)],
                                sblk)

            @pl.when(v)
            def _():
                srcv = sblk[pl.ds(cs - blk_lo2, EC)]
                pltpu.async_copy(xl_hbm.at[srcv], buf, sem)
            return blk_lo2

        def compute(st, buf, sem):
            v = valid(st)

            @pl.when(v)
            def _():
                pltpu.make_async_copy(xl_hbm.at[pl.ds(0, EC)], buf, sem).wait()

            if True:
                dl, k, ss, ln = st
                dl = jnp.minimum(dl, DG - 1)
                accs = [jnp.zeros((16,), jnp.float32) for _ in range(EC)]

                def chan_body(c, accs):
                    ch = pl.ds(c * H, H)
                    att_c = attv[ch]
                    xi_c = xrg[dl, ch]
                    out = []
                    for e in range(EC):
                        u = xi_c + buf[e, ch]
                        out.append(accs[e] + att_c * jnp.abs(u))
                    return out
                accs = lax.fori_loop(0, C, chan_body, accs)

                arv = xrg[dl, pl.ds(C * H, H)]
                rem = ln - k * EC
                ps = []
                for e in range(EC):
                    l_e = accs[e] + (buf[e, pl.ds(C * H, H)] + arv)
                    l_e = jnp.minimum(jnp.maximum(l_e, -CLAMP), CLAMP)
                    p_e = jnp.exp(l_e)
                    ps.append(jnp.where(rem > e, p_e, jnp.zeros_like(p_e)))
                dsum = ps[0]
                for e in range(1, EC):
                    dsum = dsum + ps[e]
                deng[dl, :] = deng[dl, :] + dsum

                def agg_body(c, _):
                    ch = pl.ds(c * H, H)
                    o = outg[dl, ch]
                    for e in range(EC):
                        o = o + ps[e] * buf[e, ch]
                    outg[dl, ch] = o
                    return 0
                lax.fori_loop(0, C, agg_body, 0)

        def trip_body(dl, t):
            _, ln = seg_at(dl)
            return t + jnp.maximum((ln + EC - 1) // EC, 1)

        trip = lax.fori_loop(0, DG, trip_body, jnp.int32(0))

        bufs = (xjb0, xjb1)
        sems = (gsem0, gsem1)
        ss0, ln0 = seg_at(jnp.int32(0))
        st = (jnp.int32(0), jnp.int32(0), ss0, ln0)
        blk_lo = jnp.int32(-2**30)
        sts = []
        for r in range(2):
            sts.append(st)
            blk_lo = issue(st, blk_lo, bufs[r], sems[r])
            st = advance(st)

        def body(q, carry):
            st0, st1, stn, blk_lo = carry
            sts = [st0, st1]
            for r in range(2):
                compute(sts[r], bufs[r], sems[r])
                blk_lo = issue(stn, blk_lo, bufs[r], sems[r])
                sts[r] = stn
                stn = advance(stn)
            return (sts[0], sts[1], stn, blk_lo)

        lax.fori_loop(0, (trip + 1) // 2, body,
                      (sts[0], sts[1], st, blk_lo))

        pltpu.sync_copy(outg, num_hbm.at[pl.ds(s * N + d0, DG)])
        pltpu.sync_copy(deng, den_hbm.at[pl.ds(s * N + d0, DG)])
        return 0

    lax.fori_loop(0, ngrp, group_body, 0)


@jax.jit
def _sc_layer(xl, xr, attp, srcs, rptr):
    mesh = plsc.VectorSubcoreMesh(core_axis_name="c", subcore_axis_name="s",
                                  num_cores=NSC, num_subcores=NSUB)
    f = pl.kernel(
        _sc_layer_body,
        out_type=[
            jax.ShapeDtypeStruct((NSC * N, C * H), jnp.float32),
            jax.ShapeDtypeStruct((NSC * N, H), jnp.float32),
        ],
        mesh=mesh,
        scratch_types=[
            pltpu.VMEM((C * H,), jnp.float32),
            pltpu.VMEM((656,), jnp.int32),
            pltpu.VMEM((DG, TW), jnp.float32),
            pltpu.VMEM((EC, TW), jnp.float32),
            pltpu.VMEM((EC, TW), jnp.float32),
            pltpu.VMEM((DG, C * H), jnp.float32),
            pltpu.VMEM((DG, H), jnp.float32),
            pltpu.VMEM((SBLK,), jnp.int32),
            pltpu.SemaphoreType.DMA,
            pltpu.SemaphoreType.DMA,
        ],
    )
    return f(xl, xr, attp, srcs, rptr)


BN = 400
TGRID = N // BN


def _adot_table(v, att_row):
    rows = jax.lax.broadcasted_iota(jnp.int32, (H * C, H), 0) % H
    colh = jax.lax.broadcasted_iota(jnp.int32, (H * C, H), 1)
    T2 = (rows == colh).astype(jnp.float32)
    a = jnp.dot(v * att_row, T2, preferred_element_type=jnp.float32) * 0.6
    pad = jnp.zeros((v.shape[0], TW - H * C - H), jnp.float32)
    return jnp.concatenate([v, a, pad], axis=1)


def _tc1_body(x_ref, wl_ref, bl_ref, wr_ref, br_ref, att_ref,
              xl_ref, xr_ref):
    x = x_ref[...]
    att_row = att_ref[...]
    xl = jnp.dot(x, wl_ref[...],
                 preferred_element_type=jnp.float32) + bl_ref[...]
    xr = jnp.dot(x, wr_ref[...],
                 preferred_element_type=jnp.float32) + br_ref[...]
    xl_ref[...] = _adot_table(xl, att_row)
    xr_ref[...] = _adot_table(xr, att_row)


def _tc1(x, Wlp, blp, Wrp, brp, att_row):
    f_in = x.shape[1]
    return pl.pallas_call(
        _tc1_body,
        grid=(TGRID,),
        in_specs=[
            pl.BlockSpec((BN, f_in), lambda i: (i, 0)),
            pl.BlockSpec((f_in, H * C), lambda i: (0, 0)),
            pl.BlockSpec((1, H * C), lambda i: (0, 0)),
            pl.BlockSpec((f_in, H * C), lambda i: (0, 0)),
            pl.BlockSpec((1, H * C), lambda i: (0, 0)),
            pl.BlockSpec((1, H * C), lambda i: (0, 0)),
        ],
        out_specs=[
            pl.BlockSpec((BN, TW), lambda i: (i, 0)),
            pl.BlockSpec((BN, TW), lambda i: (i, 0)),
        ],
        out_shape=[
            jax.ShapeDtypeStruct((N, TW), jnp.float32),
            jax.ShapeDtypeStruct((N, TW), jnp.float32),
        ],
    )(x, Wlp, blp[None, :], Wrp, brp[None, :], att_row[None, :])


def _combine_h(n0, n1, d0, d1, bias):
    dsum = d0 + d1 + 1e-16
    col = jax.lax.broadcasted_iota(jnp.int32, (H, H * C), 1) % H
    rowh = jax.lax.broadcasted_iota(jnp.int32, (H, H * C), 0)
    T = (col == rowh).astype(jnp.float32)
    den_b = jnp.dot(dsum, T, preferred_element_type=jnp.float32)
    hfull = (n0 + n1) / den_b
    rows = jax.lax.broadcasted_iota(jnp.int32, (H * C, C), 0) // H
    colc = jax.lax.broadcasted_iota(jnp.int32, (H * C, C), 1)
    S = (rows == colc).astype(jnp.float32)
    hm = jnp.dot(hfull, S, preferred_element_type=jnp.float32) * (1.0 / H)
    hb = hm + bias
    return jnp.maximum(hb, 0.0) + 0.01 * jnp.minimum(hb, 0.0)


def _tc2_body(n0_ref, n1_ref, d0_ref, d1_ref, b_ref, wl_ref, bl_ref,
              wr_ref, br_ref, att_ref, xl_ref, xr_ref):
    h1 = _combine_h(n0_ref[...], n1_ref[...], d0_ref[...], d1_ref[...],
                    b_ref[...])
    att_row = att_ref[...]
    xl = jnp.dot(h1, wl_ref[...],
                 preferred_element_type=jnp.float32) + bl_ref[...]
    xr = jnp.dot(h1, wr_ref[...],
                 preferred_element_type=jnp.float32) + br_ref[...]
    xl_ref[...] = _adot_table(xl, att_row)
    xr_ref[...] = _adot_table(xr, att_row)


def _tc2(num, den, bias1, Wl2p, bl2p, Wr2p, br2p, att_row):
    return pl.pallas_call(
        _tc2_body,
        grid=(TGRID,),
        in_specs=[
            pl.BlockSpec((BN, H * C), lambda i: (i, 0)),
            pl.BlockSpec((BN, H * C), lambda i: (TGRID + i, 0)),
            pl.BlockSpec((BN, H), lambda i: (i, 0)),
            pl.BlockSpec((BN, H), lambda i: (TGRID + i, 0)),
            pl.BlockSpec((1, C), lambda i: (0, 0)),
            pl.BlockSpec((C, H * C), lambda i: (0, 0)),
            pl.BlockSpec((1, H * C), lambda i: (0, 0)),
            pl.BlockSpec((C, H * C), lambda i: (0, 0)),
            pl.BlockSpec((1, H * C), lambda i: (0, 0)),
            pl.BlockSpec((1, H * C), lambda i: (0, 0)),
        ],
        out_specs=[
            pl.BlockSpec((BN, TW), lambda i: (i, 0)),
            pl.BlockSpec((BN, TW), lambda i: (i, 0)),
        ],
        out_shape=[
            jax.ShapeDtypeStruct((N, TW), jnp.float32),
            jax.ShapeDtypeStruct((N, TW), jnp.float32),
        ],
    )(num, num, den, den, bias1[None, :], Wl2p, bl2p[None, :], Wr2p,
      br2p[None, :], att_row[None, :])


def _tc3_body(n0_ref, n1_ref, d0_ref, d1_ref, b_ref, batch_ref, wc_ref,
              bc_ref, o_ref, pool_ref, cnt_ref):
    i = pl.program_id(0)
    h2 = _combine_h(n0_ref[...], n1_ref[...], d0_ref[...], d1_ref[...],
                    b_ref[...])
    b = batch_ref[0, 0, :]
    g = jax.lax.broadcasted_iota(jnp.int32, (NGRAPH, BN), 0)
    oh = (b[None, :] == g).astype(jnp.float32)
    pool = jnp.dot(oh, h2, preferred_element_type=jnp.float32)
    cnt = jnp.sum(oh, axis=1, keepdims=True)

    @pl.when(i == 0)
    def _():
        pool_ref[...] = jnp.zeros_like(pool_ref)
        cnt_ref[...] = jnp.zeros_like(cnt_ref)

    pool_ref[...] += pool
    cnt_ref[...] += cnt

    @pl.when(i == TGRID - 1)
    def _():
        pooled = pool_ref[...] / jnp.maximum(cnt_ref[...], 1.0)
        o_ref[...] = jnp.dot(pooled, wc_ref[...],
                             preferred_element_type=jnp.float32) + bc_ref[...]


def _tc3(num, den, bias2, batch3, Wc, bc):
    return pl.pallas_call(
        _tc3_body,
        grid=(TGRID,),
        in_specs=[
            pl.BlockSpec((BN, H * C), lambda i: (i, 0)),
            pl.BlockSpec((BN, H * C), lambda i: (TGRID + i, 0)),
            pl.BlockSpec((BN, H), lambda i: (i, 0)),
            pl.BlockSpec((BN, H), lambda i: (TGRID + i, 0)),
            pl.BlockSpec((1, C), lambda i: (0, 0)),
            pl.BlockSpec((1, 1, BN), lambda i: (i, 0, 0)),
            pl.BlockSpec((C, NCLASS), lambda i: (0, 0)),
            pl.BlockSpec((1, NCLASS), lambda i: (0, 0)),
        ],
        out_specs=pl.BlockSpec((NGRAPH, NCLASS), lambda i: (0, 0)),
        out_shape=jax.ShapeDtypeStruct((NGRAPH, NCLASS), jnp.float32),
        scratch_shapes=[
            pltpu.VMEM((NGRAPH, C), jnp.float32),
            pltpu.VMEM((NGRAPH, 1), jnp.float32),
        ],
    )(num, num, den, den, bias2[None, :], batch3, Wc, bc[None, :])


def _perm_w(Wl, bl):
    Wlp = Wl.reshape(-1, H, C).transpose(0, 2, 1).reshape(-1, H * C)
    blp = bl.reshape(H, C).T.reshape(-1)
    return Wlp, blp


def _prep_edges(edge_index):
    keys = jnp.sort(edge_index[1] * 16384 + edge_index[0])
    srcs = keys & 16383
    bounds = jnp.arange(N + 1) * 16384
    h0 = jnp.searchsorted(keys[:ES], bounds, side="left")
    h1 = jnp.searchsorted(keys[ES:], bounds, side="left")
    rptr = jnp.stack([h0, h1]).astype(jnp.int32)
    rptr = jnp.concatenate(
        [rptr, jnp.full((NSC, RPTR_W - (N + 1)), ES, jnp.int32)], axis=1)
    srcs2 = jnp.concatenate(
        [srcs[:ES], jnp.zeros((PAD,), jnp.int32),
         srcs[ES:], jnp.zeros((PAD,), jnp.int32)])
    return srcs2.astype(jnp.int32), rptr.reshape(-1)


def kernel(x, edge_index, batch, Wl1, bl1, Wr1, br1, att1, bias1,
           Wl2, bl2, Wr2, br2, att2, bias2, Wc, bc):
    srcs, rptr = _prep_edges(edge_index)
    Wl1p, bl1p = _perm_w(Wl1, bl1)
    Wr1p, br1p = _perm_w(Wr1, br1)
    Wl2p, bl2p = _perm_w(Wl2, bl2)
    Wr2p, br2p = _perm_w(Wr2, br2)
    att1p = att1.T.reshape(-1)
    att2p = att2.T.reshape(-1)
    xl1, xr1 = _tc1(x, Wl1p, bl1p, Wr1p, br1p, att1p)
    num1, den1 = _sc_layer(xl1, xr1, 0.4 * att1p, srcs, rptr)
    xl2, xr2 = _tc2(num1, den1, bias1, Wl2p, bl2p, Wr2p, br2p, att2p)
    num2, den2 = _sc_layer(xl2, xr2, 0.4 * att2p, srcs, rptr)
    batch3 = batch.astype(jnp.int32).reshape(TGRID, 1, BN)
    return _tc3(num2, den2, bias2, batch3, Wc, bc)
